# baseline (speedup 1.0000x reference)
import functools

import jax
import jax.numpy as jnp
from jax import lax
from jax.experimental import pallas as pl
from jax.experimental.pallas import tpu as pltpu
from jax.experimental.pallas import tpu_sc as plsc

N = 10000
E = 320000
XF = 128
H = 10
EF = 16
P = 5000
HP = 16
NC, NS = 2, 16
NW = NC * NS
BB = 80
NB = 125
EPW = NB * BB
NPAD = N
SRW = 624
TAIL = NPAD - SRW * NS
PB = 200
NPB = P // PB
NBLK = 5
BR = N // NBLK
EBLK = 40
EBR = E // EBLK
EPS = 1e-5


@functools.lru_cache(maxsize=None)
def _get_mesh():
    return plsc.VectorSubcoreMesh(core_axis_name="c", subcore_axis_name="s",
                                  num_cores=NC, num_subcores=NS)


@functools.lru_cache(maxsize=None)
def _sc_pass1():
    @functools.partial(
        pl.kernel,
        out_type=(jax.ShapeDtypeStruct((NC, NPAD, HP), jnp.float32),
                  jax.ShapeDtypeStruct((NC, NPAD, 8), jnp.float32)),
        mesh=_get_mesh(),
        compiler_params=pltpu.CompilerParams(use_tc_tiling_on_sc=False),
        scratch_types=(pltpu.VMEM((2, BB), jnp.int32),
                       pltpu.VMEM((2, BB, EF), jnp.float32),
                       pltpu.VMEM((BB, 8), jnp.float32),
                       pltpu.VMEM_SHARED((NPAD, HP), jnp.float32),
                       pltpu.VMEM_SHARED((NPAD, 8), jnp.float32),
                       pltpu.SemaphoreType.DMA,
                       pltpu.SemaphoreType.DMA),
    )
    def k(ea_hbm, ei_hbm, zeros_hbm, zeros8_hbm, ones_hbm,
          ea_out, deg_out,
          dstb_v, rows_v, ones_v, ea_s, deg_s, lsem, ssem):
        c = lax.axis_index("c")
        s = lax.axis_index("s")
        w = s * NC + c
        base = w * EPW
        pltpu.sync_copy(ones_hbm, ones_v)
        pltpu.sync_copy(zeros_hbm, ea_s.at[pl.ds(s * SRW, SRW)])
        pltpu.sync_copy(zeros8_hbm, deg_s.at[pl.ds(s * SRW, SRW)])

        @pl.when(s == NS - 1)
        def _():
            pltpu.sync_copy(zeros_hbm.at[pl.ds(0, TAIL)], ea_s.at[pl.ds(SRW * NS, TAIL)])
            pltpu.sync_copy(zeros8_hbm.at[pl.ds(0, TAIL)], deg_s.at[pl.ds(SRW * NS, TAIL)])

        plsc.subcore_barrier()
        pltpu.sync_copy(ei_hbm.at[1, pl.ds(base, BB)], dstb_v.at[0])
        pltpu.async_copy(ea_hbm.at[pl.ds(base, BB)], rows_v.at[0], lsem)

        def body(j, carry):
            b = lax.rem(j, 2)
            nb = 1 - b
            pltpu.make_async_copy(ea_hbm.at[pl.ds(base + j * BB, BB)],
                                  rows_v.at[b], lsem).wait()

            @pl.when(j >= 1)
            def _():
                pltpu.make_async_copy(rows_v.at[nb], ea_s.at[dstb_v.at[nb]], ssem).wait()
                pltpu.make_async_copy(ones_v, deg_s.at[dstb_v.at[nb]], ssem).wait()

            @pl.when(j < NB - 1)
            def _():
                pltpu.sync_copy(ei_hbm.at[1, pl.ds(base + (j + 1) * BB, BB)], dstb_v.at[nb])
                pltpu.async_copy(ea_hbm.at[pl.ds(base + (j + 1) * BB, BB)],
                                 rows_v.at[nb], lsem)

            pltpu.async_copy(rows_v.at[b], ea_s.at[dstb_v.at[b]], ssem, add=True)
            pltpu.async_copy(ones_v, deg_s.at[dstb_v.at[b]], ssem, add=True)
            return carry

        lax.fori_loop(0, NB, body, 0)
        pltpu.make_async_copy(rows_v.at[1], ea_s.at[dstb_v.at[1]], ssem).wait()
        pltpu.make_async_copy(ones_v, deg_s.at[dstb_v.at[1]], ssem).wait()
        plsc.subcore_barrier()
        pltpu.sync_copy(ea_s.at[pl.ds(s * SRW, SRW)], ea_out.at[c, pl.ds(s * SRW, SRW)])
        pltpu.sync_copy(deg_s.at[pl.ds(s * SRW, SRW)], deg_out.at[c, pl.ds(s * SRW, SRW)])

        @pl.when(s == NS - 1)
        def _():
            pltpu.sync_copy(ea_s.at[pl.ds(SRW * NS, TAIL)], ea_out.at[c, pl.ds(SRW * NS, TAIL)])
            pltpu.sync_copy(deg_s.at[pl.ds(SRW * NS, TAIL)], deg_out.at[c, pl.ds(SRW * NS, TAIL)])

    return k


@functools.lru_cache(maxsize=None)
def _sc_segsum():
    @functools.partial(
        pl.kernel,
        out_type=jax.ShapeDtypeStruct((NC, NPAD, HP), jnp.float32),
        mesh=_get_mesh(),
        compiler_params=pltpu.CompilerParams(use_tc_tiling_on_sc=False),
        scratch_types=(pltpu.VMEM((EPW,), jnp.int32),
                       pltpu.VMEM((4, BB), jnp.int32),
                       pltpu.VMEM((4, BB, HP), jnp.float32),
                       pltpu.VMEM_SHARED((NPAD, HP), jnp.float32),
                       pltpu.VMEM_SHARED((NPAD, HP), jnp.float32),
                       pltpu.SemaphoreType.DMA,
                       pltpu.SemaphoreType.DMA),
    )
    def k(y_hbm, ei_hbm, zeros_hbm,
          out_hbm,
          src_f, dstb_v, rows_v, agg_s, y_s, gsem, ssem):
        c = lax.axis_index("c")
        s = lax.axis_index("s")
        w = s * NC + c
        base = w * EPW
        pltpu.sync_copy(ei_hbm.at[0, pl.ds(base, EPW)], src_f)
        pltpu.sync_copy(zeros_hbm, agg_s.at[pl.ds(s * SRW, SRW)])
        pltpu.sync_copy(y_hbm.at[pl.ds(s * SRW, SRW)], y_s.at[pl.ds(s * SRW, SRW)])

        @pl.when(s == NS - 1)
        def _():
            pltpu.sync_copy(zeros_hbm.at[pl.ds(0, TAIL)], agg_s.at[pl.ds(SRW * NS, TAIL)])
            pltpu.sync_copy(y_hbm.at[pl.ds(SRW * NS, TAIL)], y_s.at[pl.ds(SRW * NS, TAIL)])

        plsc.subcore_barrier()
        for t in range(3):
            pltpu.sync_copy(ei_hbm.at[1, pl.ds(base + t * BB, BB)], dstb_v.at[t])
            pltpu.async_copy(y_s.at[src_f.at[pl.ds(t * BB, BB)]], rows_v.at[t], gsem)

        def body(j, carry):
            b = lax.rem(j, 4)
            nxt = lax.rem(j + 3, 4)
            pltpu.make_async_copy(y_s.at[src_f.at[pl.ds(j * BB, BB)]],
                                  rows_v.at[b], gsem).wait()

            @pl.when(j >= 1)
            def _():
                pltpu.make_async_copy(rows_v.at[nxt],
                                      agg_s.at[dstb_v.at[nxt]], ssem).wait()

            @pl.when(j + 3 < NB)
            def _():
                pltpu.sync_copy(ei_hbm.at[1, pl.ds(base + (j + 3) * BB, BB)], dstb_v.at[nxt])
                pltpu.async_copy(y_s.at[src_f.at[pl.ds((j + 3) * BB, BB)]],
                                 rows_v.at[nxt], gsem)

            pltpu.async_copy(rows_v.at[b], agg_s.at[dstb_v.at[b]], ssem, add=True)
            return carry

        lax.fori_loop(0, NB, body, 0)
        pltpu.make_async_copy(rows_v.at[lax.rem(NB - 1, 4)],
                              agg_s.at[dstb_v.at[lax.rem(NB - 1, 4)]], ssem).wait()
        plsc.subcore_barrier()
        pltpu.sync_copy(agg_s.at[pl.ds(s * SRW, SRW)], out_hbm.at[c, pl.ds(s * SRW, SRW)])

        @pl.when(s == NS - 1)
        def _():
            pltpu.sync_copy(agg_s.at[pl.ds(SRW * NS, TAIL)], out_hbm.at[c, pl.ds(SRW * NS, TAIL)])

    return k


@functools.lru_cache(maxsize=None)
def _sc_pick():
    @functools.partial(
        pl.kernel,
        out_type=jax.ShapeDtypeStruct((P, 8), jnp.float32),
        mesh=_get_mesh(),
        compiler_params=pltpu.CompilerParams(use_tc_tiling_on_sc=False),
        scratch_types=(pltpu.VMEM((2, PB // 2), jnp.int32),
                       pltpu.VMEM((PB, 8), jnp.float32),
                       pltpu.SemaphoreType.DMA),
    )
    def k(probs_hbm, idx3_hbm, out_hbm, idx_v, rows_v, sem):
        c = lax.axis_index("c")
        s = lax.axis_index("s")
        w = s * NC + c

        @pl.when(w < NPB)
        def _():
            pltpu.sync_copy(idx3_hbm.at[w], idx_v)
            pltpu.async_copy(probs_hbm.at[idx_v.at[0]],
                             rows_v.at[pl.ds(0, PB // 2)], sem).wait()
            pltpu.async_copy(probs_hbm.at[idx_v.at[1]],
                             rows_v.at[pl.ds(PB // 2, PB // 2)], sem).wait()
            pltpu.sync_copy(rows_v, out_hbm.at[pl.ds(w * PB, PB)])

    return k


def _k1_body(x_ref, w0_ref, b0_ref, w80_ref, x1a_ref, p_ref, st_ref):
    i = pl.program_id(0)
    xb = x_ref[...]
    x1 = jnp.dot(xb, w0_ref[...], preferred_element_type=jnp.float32) + b0_ref[...]
    x1a_ref[...] = x1
    p_ref[...] = jnp.dot(xb, w80_ref[...], preferred_element_type=jnp.float32)
    s0 = jnp.sum(x1, axis=0, keepdims=True)
    s1 = jnp.sum(x1 * x1, axis=0, keepdims=True)
    st = jnp.concatenate([s0, s1, jnp.zeros((6, XF), jnp.float32)], axis=0)

    @pl.when(i == 0)
    def _():
        st_ref[...] = st

    @pl.when(i > 0)
    def _():
        st_ref[...] += st


def _k1(x, w0, b0, w80):
    return pl.pallas_call(
        _k1_body,
        grid=(NBLK,),
        in_specs=[pl.BlockSpec((BR, XF), lambda i: (i, 0)),
                  pl.BlockSpec((XF, XF), lambda i: (0, 0)),
                  pl.BlockSpec((1, XF), lambda i: (0, 0)),
                  pl.BlockSpec((XF, 128), lambda i: (0, 0))],
        out_specs=[pl.BlockSpec((BR, XF), lambda i: (i, 0)),
                   pl.BlockSpec((BR, 128), lambda i: (i, 0)),
                   pl.BlockSpec((8, XF), lambda i: (0, 0))],
        out_shape=[jax.ShapeDtypeStruct((N, XF), jnp.float32),
                   jax.ShapeDtypeStruct((N, 128), jnp.float32),
                   jax.ShapeDtypeStruct((8, XF), jnp.float32)],
    )(x, w0, b0, w80)


def _k2_body(x1a_ref, st_ref, p_ref, ng_ref, nb_ref, na_ref, w02_ref, b02_ref,
             wmr1_ref, r0_ref, y1_ref):
    st = st_ref[...]
    mean = st[0:1, :] / N
    ex2 = st[1:2, :] / N
    al = na_ref[...]
    var = ex2 - (2.0 * al - al * al) * mean * mean
    sg = ng_ref[...] * lax.rsqrt(var + EPS)
    tg = nb_ref[...] - sg * al * mean
    x1n = x1a_ref[...] * sg + tg
    r0 = jnp.maximum(jnp.dot(x1n, w02_ref[...], preferred_element_type=jnp.float32)
                     + b02_ref[...], 0.0)
    r0_ref[...] = r0
    y1_ref[...] = p_ref[...][:, 0:HP] + jnp.dot(r0, wmr1_ref[...],
                                                preferred_element_type=jnp.float32)


def _k2(x1a, st, p, ng, nb, na, w02, b02, wmr1):
    return pl.pallas_call(
        _k2_body,
        grid=(NBLK,),
        in_specs=[pl.BlockSpec((BR, XF), lambda i: (i, 0)),
                  pl.BlockSpec((8, XF), lambda i: (0, 0)),
                  pl.BlockSpec((BR, 128), lambda i: (i, 0)),
                  pl.BlockSpec((1, XF), lambda i: (0, 0)),
                  pl.BlockSpec((1, XF), lambda i: (0, 0)),
                  pl.BlockSpec((1, XF), lambda i: (0, 0)),
                  pl.BlockSpec((XF, HP), lambda i: (0, 0)),
                  pl.BlockSpec((1, HP), lambda i: (0, 0)),
                  pl.BlockSpec((HP, HP), lambda i: (0, 0))],
        out_specs=[pl.BlockSpec((BR, HP), lambda i: (i, 0)),
                   pl.BlockSpec((BR, HP), lambda i: (i, 0))],
        out_shape=[jax.ShapeDtypeStruct((N, HP), jnp.float32),
                   jax.ShapeDtypeStruct((NPAD, HP), jnp.float32)],
    )(x1a, st, p, ng, nb, na, w02, b02, wmr1)


def _k3_body(x_ref, ca_ref):
    i = pl.program_id(0)
    xb = x_ref[...]
    mm = lax.dot_general(xb, xb, (((0,), (0,)), ((), ())),
                         preferred_element_type=jnp.float32)
    ms = jnp.sum(xb, axis=0, keepdims=True)
    c16 = mm[0:EF, 0:EF]
    m16 = ms[:, 0:EF]
    for g in range(1, 8):
        c16 = c16 + mm[g * EF:(g + 1) * EF, g * EF:(g + 1) * EF]
        m16 = m16 + ms[:, g * EF:(g + 1) * EF]
    ca = jnp.concatenate(
        [jnp.concatenate([c16, jnp.zeros((EF, 16), jnp.float32)], axis=1),
         jnp.concatenate([m16, jnp.zeros((1, 16), jnp.float32)], axis=1),
         jnp.zeros((15, 32), jnp.float32)], axis=0)

    @pl.when(i == 0)
    def _():
        ca_ref[...] = ca

    @pl.when(i > 0)
    def _():
        ca_ref[...] += ca


def _k3(ea128):
    return pl.pallas_call(
        _k3_body,
        grid=(EBLK,),
        in_specs=[pl.BlockSpec((E // 8 // EBLK, 128), lambda i: (i, 0))],
        out_specs=pl.BlockSpec((32, 32), lambda i: (0, 0)),
        out_shape=jax.ShapeDtypeStruct((32, 32), jnp.float32),
    )(ea128)


def _k4_body(eap_ref, degp_ref, ca_ref, p_ref, w0e_ref, b0e_ref,
             n2g_ref, n2b_ref, n2a_ref, wme_ref, bm4_ref, br4_ref,
             const_ref):
    ca = ca_ref[...]
    m = ca[16:17, 0:EF] / E
    cmat = ca[0:EF, 0:EF] / E
    w = w0e_ref[...]
    b = b0e_ref[...]
    mw = jnp.dot(m, w, preferred_element_type=jnp.float32)
    mu = mw + b
    cw = jnp.dot(cmat, w, preferred_element_type=jnp.float32)
    eu2 = jnp.sum(w * cw, axis=0, keepdims=True) + 2.0 * b * mw + b * b
    al = n2a_ref[...]
    var = eu2 - (2.0 * al - al * al) * mu * mu
    sg = n2g_ref[...] * lax.rsqrt(var + EPS)
    tg = n2b_ref[...] - sg * al * mu
    ea = eap_ref[0] + eap_ref[1]
    deg = (degp_ref[0] + degp_ref[1])[:, 0:1]
    ea_agg = (jnp.dot(ea, w, preferred_element_type=jnp.float32) + deg * b) * sg + deg * tg
    pblk = p_ref[...]
    for l in range(4):
        cl = (pblk[:, (4 + l) * HP:(5 + l) * HP]
              + jnp.dot(ea_agg, wme_ref[...][:, l * HP:(l + 1) * HP],
                        preferred_element_type=jnp.float32)
              + deg * bm4_ref[l:l + 1, :] + br4_ref[l:l + 1, :])
        const_ref[l] = cl


def _k4(eap, degp, ca, p, w0e, b0e, n2g, n2b, n2a, wme, bm4, br4):
    return pl.pallas_call(
        _k4_body,
        grid=(NBLK,),
        in_specs=[pl.BlockSpec((NC, BR, HP), lambda i: (0, i, 0)),
                  pl.BlockSpec((NC, BR, 8), lambda i: (0, i, 0)),
                  pl.BlockSpec((32, 32), lambda i: (0, 0)),
                  pl.BlockSpec((BR, 128), lambda i: (i, 0)),
                  pl.BlockSpec((EF, EF), lambda i: (0, 0)),
                  pl.BlockSpec((1, EF), lambda i: (0, 0)),
                  pl.BlockSpec((1, EF), lambda i: (0, 0)),
                  pl.BlockSpec((1, EF), lambda i: (0, 0)),
                  pl.BlockSpec((1, EF), lambda i: (0, 0)),
                  pl.BlockSpec((EF, 4 * HP), lambda i: (0, 0)),
                  pl.BlockSpec((4, HP), lambda i: (0, 0)),
                  pl.BlockSpec((4, HP), lambda i: (0, 0))],
        out_specs=pl.BlockSpec((4, BR, HP), lambda i: (0, i, 0)),
        out_shape=jax.ShapeDtypeStruct((4, N, HP), jnp.float32),
    )(eap, degp, ca, p, w0e, b0e, n2g, n2b, n2a, wme, bm4, br4)


def _k5_body(sp_ref, const_ref, r_ref, pa_ref, wrr_ref, wmr_ref,
             rn_ref, y_ref):
    rn = jnp.maximum(const_ref[...]
                     + jnp.dot(r_ref[...], wrr_ref[...], preferred_element_type=jnp.float32)
                     + sp_ref[0] + sp_ref[1], 0.0)
    rn_ref[...] = rn
    y_ref[...] = pa_ref[...] + jnp.dot(rn, wmr_ref[...],
                                       preferred_element_type=jnp.float32)


def _k5(sp, const, r, pa, wrr, wmr):
    return pl.pallas_call(
        _k5_body,
        grid=(NBLK,),
        in_specs=[pl.BlockSpec((NC, BR, HP), lambda i: (0, i, 0)),
                  pl.BlockSpec((BR, HP), lambda i: (i, 0)),
                  pl.BlockSpec((BR, HP), lambda i: (i, 0)),
                  pl.BlockSpec((BR, HP), lambda i: (i, 0)),
                  pl.BlockSpec((HP, HP), lambda i: (0, 0)),
                  pl.BlockSpec((HP, HP), lambda i: (0, 0))],
        out_specs=[pl.BlockSpec((BR, HP), lambda i: (i, 0)),
                   pl.BlockSpec((BR, HP), lambda i: (i, 0))],
        out_shape=[jax.ShapeDtypeStruct((N, HP), jnp.float32),
                   jax.ShapeDtypeStruct((NPAD, HP), jnp.float32)],
    )(sp, const, r, pa, wrr, wmr)


def _k6_body(sp_ref, const_ref, r_ref, wrr_ref, w5_ref, b5_ref, probs_ref):
    r4 = jnp.maximum(const_ref[...]
                     + jnp.dot(r_ref[...], wrr_ref[...], preferred_element_type=jnp.float32)
                     + sp_ref[0] + sp_ref[1], 0.0)
    logits = jnp.dot(r4, w5_ref[...], preferred_element_type=jnp.float32) + b5_ref[...]
    a = logits[:, 0:1]
    b = logits[:, 1:2]
    mx = jnp.maximum(a, b)
    e0 = jnp.exp(a - mx)
    e1 = jnp.exp(b - mx)
    den = e0 + e1
    probs_ref[...] = jnp.concatenate(
        [e0 / den, e1 / den, jnp.zeros((BR, 6), jnp.float32)], axis=1)


def _k6(sp, const, r, wrr, w5, b5):
    return pl.pallas_call(
        _k6_body,
        grid=(NBLK,),
        in_specs=[pl.BlockSpec((NC, BR, HP), lambda i: (0, i, 0)),
                  pl.BlockSpec((BR, HP), lambda i: (i, 0)),
                  pl.BlockSpec((BR, HP), lambda i: (i, 0)),
                  pl.BlockSpec((HP, HP), lambda i: (0, 0)),
                  pl.BlockSpec((HP, HP), lambda i: (0, 0)),
                  pl.BlockSpec((1, HP), lambda i: (0, 0))],
        out_specs=pl.BlockSpec((BR, 8), lambda i: (i, 0)),
        out_shape=jax.ShapeDtypeStruct((N, 8), jnp.float32),
    )(sp, const, r, wrr, w5, b5)


def _pad(w, rows, cols):
    return jnp.zeros((rows, cols), jnp.float32).at[:w.shape[0], :w.shape[1]].set(w)


def kernel(x, z, edge_index, z1edge_index, z2edge_index, z3edge_index,
           edge_attr, pickable, l0_W, l0_b, l02_W, l02_b, l0e_W, l0e_b,
           n_gamma, n_beta, n_alpha, n2_gamma, n2_beta, n2_alpha,
           l1_Wm, l1_bm, l1_Wr, l1_br, l2_Wm, l2_bm, l2_Wr, l2_br,
           l3_Wm, l3_bm, l3_Wr, l3_br, l4_Wm, l4_bm, l4_Wr, l4_br,
           l5_W, l5_b):
    wms = [l1_Wm, l2_Wm, l3_Wm, l4_Wm]
    wrs = [l1_Wr, l2_Wr, l3_Wr, l4_Wr]
    bms = [l1_bm, l2_bm, l3_bm, l4_bm]
    brs = [l1_br, l2_br, l3_br, l4_br]

    w80 = jnp.concatenate(
        [_pad(wm[:XF], XF, HP) for wm in wms] + [_pad(wr[:XF], XF, HP) for wr in wrs],
        axis=1)
    wmr = [_pad(wm[XF:XF + H], HP, HP) for wm in wms]
    wrr = [_pad(wr[XF:], HP, HP) for wr in wrs]
    wme = jnp.concatenate([_pad(wm[XF + H:], EF, HP) for wm in wms], axis=1)
    bm4 = jnp.concatenate([_pad(b[None, :], 1, HP) for b in bms], axis=0)
    br4 = jnp.concatenate([_pad(b[None, :], 1, HP) for b in brs], axis=0)
    w02 = _pad(l02_W, XF, HP)
    b02 = _pad(l02_b[None, :], 1, HP)
    w5 = _pad(l5_W, HP, HP)
    b5 = _pad(l5_b[None, :], 1, HP)

    pick2 = pickable.reshape(NPB, 2, PB // 2)
    zeros_s = jnp.zeros((SRW, HP), jnp.float32)
    zeros8_s = jnp.zeros((SRW, 8), jnp.float32)
    ones_b = jnp.ones((BB, 8), jnp.float32)

    x1a, p, st = _k1(x, l0_W, l0_b[None, :], w80)
    r0, y1 = _k2(x1a, st, p, n_gamma[None, :], n_beta[None, :], n_alpha[None, :],
                 w02, b02, wmr[0])
    ca = _k3(edge_attr.reshape(E // 8, 128))
    eap, degp = _sc_pass1()(edge_attr, edge_index, zeros_s, zeros8_s, ones_b)
    const = _k4(eap, degp, ca, p, l0e_W, l0e_b[None, :],
                n2_gamma[None, :], n2_beta[None, :], n2_alpha[None, :],
                wme, bm4, br4)

    r = r0
    y = y1 + eap[0, 0:1, 0:1] * 0.0
    probs = None
    for l in range(4):
        sp = _sc_segsum()(y, edge_index, zeros_s)
        if l < 3:
            pa = lax.slice(p, (0, (l + 1) * HP), (N, (l + 2) * HP))
            r, y = _k5(sp, const[l], r, pa, wrr[l], wmr[l + 1])
        else:
            probs = _k6(sp, const[l], r, wrr[l], w5, b5)

    picked = _sc_pick()(probs, pick2)
    return picked[:, 0:2]


# speedup vs baseline: 1.1248x; 1.1248x over previous
import functools

import jax
import jax.numpy as jnp
from jax import lax
from jax.experimental import pallas as pl
from jax.experimental.pallas import tpu as pltpu
from jax.experimental.pallas import tpu_sc as plsc

N = 10000
E = 320000
XF = 128
H = 10
EF = 16
P = 5000
HP = 16
NC, NS = 2, 16
NW = NC * NS
BB = 128
NB = 80
EPW = NB * BB
EP = NW * EPW
NPAD = N + 8
SRW = 624
TAIL = NPAD - SRW * NS
PB = 200
NPB = P // PB
NBLK = 5
BR = N // NBLK
EBLK = 40
EBR = E // EBLK
EPS = 1e-5


@functools.lru_cache(maxsize=None)
def _get_mesh():
    return plsc.VectorSubcoreMesh(core_axis_name="c", subcore_axis_name="s",
                                  num_cores=NC, num_subcores=NS)


@functools.lru_cache(maxsize=None)
def _sc_pass1():
    @functools.partial(
        pl.kernel,
        out_type=(jax.ShapeDtypeStruct((NC, NPAD, HP), jnp.float32),
                  jax.ShapeDtypeStruct((NC, NPAD, 8), jnp.float32)),
        mesh=_get_mesh(),
        compiler_params=pltpu.CompilerParams(use_tc_tiling_on_sc=False),
        scratch_types=(pltpu.VMEM((NB, BB), jnp.int32),
                       pltpu.VMEM((2, BB, EF), jnp.float32),
                       pltpu.VMEM((BB, 8), jnp.float32),
                       pltpu.VMEM_SHARED((NPAD, HP), jnp.float32),
                       pltpu.VMEM_SHARED((NPAD, 8), jnp.float32),
                       pltpu.SemaphoreType.DMA,
                       pltpu.SemaphoreType.DMA),
    )
    def k(ea_hbm, dst3_hbm, zeros_hbm, zeros8_hbm, ones_hbm,
          ea_out, deg_out,
          dst_v, rows_v, ones_v, ea_s, deg_s, lsem, ssem):
        c = lax.axis_index("c")
        s = lax.axis_index("s")
        w = s * NC + c
        base = w * EPW
        pltpu.sync_copy(dst3_hbm.at[w], dst_v)
        pltpu.sync_copy(ones_hbm, ones_v)
        pltpu.sync_copy(zeros_hbm, ea_s.at[pl.ds(s * SRW, SRW)])
        pltpu.sync_copy(zeros8_hbm, deg_s.at[pl.ds(s * SRW, SRW)])

        @pl.when(s == NS - 1)
        def _():
            pltpu.sync_copy(zeros_hbm.at[pl.ds(0, TAIL)], ea_s.at[pl.ds(SRW * NS, TAIL)])
            pltpu.sync_copy(zeros8_hbm.at[pl.ds(0, TAIL)], deg_s.at[pl.ds(SRW * NS, TAIL)])

        plsc.subcore_barrier()

        def eoff(j):
            o = base + j * BB
            return lax.select(o <= E - BB, o, 0)

        pltpu.async_copy(ea_hbm.at[pl.ds(eoff(0), BB)], rows_v.at[0], lsem)

        def body(j, carry):
            b = lax.rem(j, 2)
            nb = 1 - b
            pltpu.make_async_copy(ea_hbm.at[pl.ds(eoff(j), BB)],
                                  rows_v.at[b], lsem).wait()

            @pl.when(j >= 1)
            def _():
                pltpu.make_async_copy(rows_v.at[nb], ea_s.at[dst_v.at[j]], ssem).wait()
                pltpu.make_async_copy(ones_v, deg_s.at[dst_v.at[j]], ssem).wait()

            @pl.when(j < NB - 1)
            def _():
                pltpu.async_copy(ea_hbm.at[pl.ds(eoff(j + 1), BB)],
                                 rows_v.at[nb], lsem)

            pltpu.async_copy(rows_v.at[b], ea_s.at[dst_v.at[j]], ssem, add=True)
            pltpu.async_copy(ones_v, deg_s.at[dst_v.at[j]], ssem, add=True)
            return carry

        lax.fori_loop(0, NB, body, 0)
        pltpu.make_async_copy(rows_v.at[1], ea_s.at[dst_v.at[NB - 1]], ssem).wait()
        pltpu.make_async_copy(ones_v, deg_s.at[dst_v.at[NB - 1]], ssem).wait()
        plsc.subcore_barrier()
        pltpu.sync_copy(ea_s.at[pl.ds(s * SRW, SRW)], ea_out.at[c, pl.ds(s * SRW, SRW)])
        pltpu.sync_copy(deg_s.at[pl.ds(s * SRW, SRW)], deg_out.at[c, pl.ds(s * SRW, SRW)])

        @pl.when(s == NS - 1)
        def _():
            pltpu.sync_copy(ea_s.at[pl.ds(SRW * NS, TAIL)], ea_out.at[c, pl.ds(SRW * NS, TAIL)])
            pltpu.sync_copy(deg_s.at[pl.ds(SRW * NS, TAIL)], deg_out.at[c, pl.ds(SRW * NS, TAIL)])

    return k


@functools.lru_cache(maxsize=None)
def _sc_segsum():
    @functools.partial(
        pl.kernel,
        out_type=jax.ShapeDtypeStruct((NC, NPAD, HP), jnp.float32),
        mesh=_get_mesh(),
        compiler_params=pltpu.CompilerParams(use_tc_tiling_on_sc=False),
        scratch_types=(pltpu.VMEM((NB, BB), jnp.int32),
                       pltpu.VMEM((NB, BB), jnp.int32),
                       pltpu.VMEM((4, BB, HP), jnp.float32),
                       pltpu.VMEM_SHARED((NPAD, HP), jnp.float32),
                       pltpu.SemaphoreType.DMA,
                       pltpu.SemaphoreType.DMA),
    )
    def k(y_hbm, src3_hbm, dst3_hbm, zeros_hbm,
          out_hbm,
          src_v, dst_v, rows_v, agg_s, gsem, ssem):
        c = lax.axis_index("c")
        s = lax.axis_index("s")
        w = s * NC + c
        pltpu.sync_copy(src3_hbm.at[w], src_v)
        pltpu.sync_copy(dst3_hbm.at[w], dst_v)
        pltpu.sync_copy(zeros_hbm, agg_s.at[pl.ds(s * SRW, SRW)])

        @pl.when(s == NS - 1)
        def _():
            pltpu.sync_copy(zeros_hbm.at[pl.ds(0, TAIL)], agg_s.at[pl.ds(SRW * NS, TAIL)])

        plsc.subcore_barrier()
        pltpu.async_copy(y_hbm.at[src_v.at[0]], rows_v.at[0], gsem)
        pltpu.async_copy(y_hbm.at[src_v.at[1]], rows_v.at[1], gsem)
        pltpu.async_copy(y_hbm.at[src_v.at[2]], rows_v.at[2], gsem)

        def body(j, carry):
            b = lax.rem(j, 4)
            pltpu.make_async_copy(y_hbm.at[src_v.at[j]], rows_v.at[b], gsem).wait()

            @pl.when(j >= 1)
            def _():
                pltpu.make_async_copy(rows_v.at[lax.rem(j + 3, 4)],
                                      agg_s.at[dst_v.at[j]], ssem).wait()

            @pl.when(j + 3 < NB)
            def _():
                pltpu.async_copy(y_hbm.at[src_v.at[j + 3]],
                                 rows_v.at[lax.rem(j + 3, 4)], gsem)

            pltpu.async_copy(rows_v.at[b], agg_s.at[dst_v.at[j]], ssem, add=True)
            return carry

        lax.fori_loop(0, NB, body, 0)
        pltpu.make_async_copy(rows_v.at[lax.rem(NB - 1, 4)],
                              agg_s.at[dst_v.at[NB - 1]], ssem).wait()
        plsc.subcore_barrier()
        pltpu.sync_copy(agg_s.at[pl.ds(s * SRW, SRW)], out_hbm.at[c, pl.ds(s * SRW, SRW)])

        @pl.when(s == NS - 1)
        def _():
            pltpu.sync_copy(agg_s.at[pl.ds(SRW * NS, TAIL)], out_hbm.at[c, pl.ds(SRW * NS, TAIL)])

    return k


@functools.lru_cache(maxsize=None)
def _sc_pick():
    @functools.partial(
        pl.kernel,
        out_type=jax.ShapeDtypeStruct((P, 8), jnp.float32),
        mesh=_get_mesh(),
        compiler_params=pltpu.CompilerParams(use_tc_tiling_on_sc=False),
        scratch_types=(pltpu.VMEM((2, PB // 2), jnp.int32),
                       pltpu.VMEM((PB, 8), jnp.float32),
                       pltpu.SemaphoreType.DMA),
    )
    def k(probs_hbm, idx3_hbm, out_hbm, idx_v, rows_v, sem):
        c = lax.axis_index("c")
        s = lax.axis_index("s")
        w = s * NC + c

        @pl.when(w < NPB)
        def _():
            pltpu.sync_copy(idx3_hbm.at[w], idx_v)
            pltpu.async_copy(probs_hbm.at[idx_v.at[0]],
                             rows_v.at[pl.ds(0, PB // 2)], sem).wait()
            pltpu.async_copy(probs_hbm.at[idx_v.at[1]],
                             rows_v.at[pl.ds(PB // 2, PB // 2)], sem).wait()
            pltpu.sync_copy(rows_v, out_hbm.at[pl.ds(w * PB, PB)])

    return k


def _k1_body(x_ref, w0_ref, b0_ref, w80_ref, x1a_ref, p_ref, st_ref):
    i = pl.program_id(0)
    xb = x_ref[...]
    x1 = jnp.dot(xb, w0_ref[...], preferred_element_type=jnp.float32) + b0_ref[...]
    x1a_ref[...] = x1
    p_ref[...] = jnp.dot(xb, w80_ref[...], preferred_element_type=jnp.float32)
    s0 = jnp.sum(x1, axis=0, keepdims=True)
    s1 = jnp.sum(x1 * x1, axis=0, keepdims=True)
    st = jnp.concatenate([s0, s1, jnp.zeros((6, XF), jnp.float32)], axis=0)

    @pl.when(i == 0)
    def _():
        st_ref[...] = st

    @pl.when(i > 0)
    def _():
        st_ref[...] += st


def _k1(x, w0, b0, w80):
    return pl.pallas_call(
        _k1_body,
        grid=(NBLK,),
        in_specs=[pl.BlockSpec((BR, XF), lambda i: (i, 0)),
                  pl.BlockSpec((XF, XF), lambda i: (0, 0)),
                  pl.BlockSpec((1, XF), lambda i: (0, 0)),
                  pl.BlockSpec((XF, 128), lambda i: (0, 0))],
        out_specs=[pl.BlockSpec((BR, XF), lambda i: (i, 0)),
                   pl.BlockSpec((BR, 128), lambda i: (i, 0)),
                   pl.BlockSpec((8, XF), lambda i: (0, 0))],
        out_shape=[jax.ShapeDtypeStruct((N, XF), jnp.float32),
                   jax.ShapeDtypeStruct((N, 128), jnp.float32),
                   jax.ShapeDtypeStruct((8, XF), jnp.float32)],
    )(x, w0, b0, w80)


def _k2_body(x1a_ref, st_ref, p_ref, ng_ref, nb_ref, na_ref, w02_ref, b02_ref,
             wmr1_ref, r0_ref, y1_ref):
    st = st_ref[...]
    mean = st[0:1, :] / N
    ex2 = st[1:2, :] / N
    al = na_ref[...]
    var = ex2 - (2.0 * al - al * al) * mean * mean
    sg = ng_ref[...] * lax.rsqrt(var + EPS)
    tg = nb_ref[...] - sg * al * mean
    x1n = x1a_ref[...] * sg + tg
    r0 = jnp.maximum(jnp.dot(x1n, w02_ref[...], preferred_element_type=jnp.float32)
                     + b02_ref[...], 0.0)
    r0_ref[...] = r0
    y1_ref[...] = p_ref[...][:, 0:HP] + jnp.dot(r0, wmr1_ref[...],
                                                preferred_element_type=jnp.float32)


def _k2(x1a, st, p, ng, nb, na, w02, b02, wmr1):
    return pl.pallas_call(
        _k2_body,
        grid=(NBLK,),
        in_specs=[pl.BlockSpec((BR, XF), lambda i: (i, 0)),
                  pl.BlockSpec((8, XF), lambda i: (0, 0)),
                  pl.BlockSpec((BR, 128), lambda i: (i, 0)),
                  pl.BlockSpec((1, XF), lambda i: (0, 0)),
                  pl.BlockSpec((1, XF), lambda i: (0, 0)),
                  pl.BlockSpec((1, XF), lambda i: (0, 0)),
                  pl.BlockSpec((XF, HP), lambda i: (0, 0)),
                  pl.BlockSpec((1, HP), lambda i: (0, 0)),
                  pl.BlockSpec((HP, HP), lambda i: (0, 0))],
        out_specs=[pl.BlockSpec((BR, HP), lambda i: (i, 0)),
                   pl.BlockSpec((BR, HP), lambda i: (i, 0))],
        out_shape=[jax.ShapeDtypeStruct((N, HP), jnp.float32),
                   jax.ShapeDtypeStruct((NPAD, HP), jnp.float32)],
    )(x1a, st, p, ng, nb, na, w02, b02, wmr1)


def _k3_body(x_ref, ca_ref):
    i = pl.program_id(0)
    xb = x_ref[...]
    mm = lax.dot_general(xb, xb, (((0,), (0,)), ((), ())),
                         preferred_element_type=jnp.float32)
    ms = jnp.sum(xb, axis=0, keepdims=True)
    c16 = mm[0:EF, 0:EF]
    m16 = ms[:, 0:EF]
    for g in range(1, 8):
        c16 = c16 + mm[g * EF:(g + 1) * EF, g * EF:(g + 1) * EF]
        m16 = m16 + ms[:, g * EF:(g + 1) * EF]
    ca = jnp.concatenate(
        [jnp.concatenate([c16, jnp.zeros((EF, 16), jnp.float32)], axis=1),
         jnp.concatenate([m16, jnp.zeros((1, 16), jnp.float32)], axis=1),
         jnp.zeros((15, 32), jnp.float32)], axis=0)

    @pl.when(i == 0)
    def _():
        ca_ref[...] = ca

    @pl.when(i > 0)
    def _():
        ca_ref[...] += ca


def _k3(ea128):
    return pl.pallas_call(
        _k3_body,
        grid=(EBLK,),
        in_specs=[pl.BlockSpec((E // 8 // EBLK, 128), lambda i: (i, 0))],
        out_specs=pl.BlockSpec((32, 32), lambda i: (0, 0)),
        out_shape=jax.ShapeDtypeStruct((32, 32), jnp.float32),
    )(ea128)


def _k4_body(eap_ref, degp_ref, ca_ref, p_ref, w0e_ref, b0e_ref,
             n2g_ref, n2b_ref, n2a_ref, wme_ref, bm4_ref, br4_ref,
             const_ref):
    ca = ca_ref[...]
    m = ca[16:17, 0:EF] / E
    cmat = ca[0:EF, 0:EF] / E
    w = w0e_ref[...]
    b = b0e_ref[...]
    mw = jnp.dot(m, w, preferred_element_type=jnp.float32)
    mu = mw + b
    cw = jnp.dot(cmat, w, preferred_element_type=jnp.float32)
    eu2 = jnp.sum(w * cw, axis=0, keepdims=True) + 2.0 * b * mw + b * b
    al = n2a_ref[...]
    var = eu2 - (2.0 * al - al * al) * mu * mu
    sg = n2g_ref[...] * lax.rsqrt(var + EPS)
    tg = n2b_ref[...] - sg * al * mu
    ea = eap_ref[0] + eap_ref[1]
    deg = (degp_ref[0] + degp_ref[1])[:, 0:1]
    ea_agg = (jnp.dot(ea, w, preferred_element_type=jnp.float32) + deg * b) * sg + deg * tg
    pblk = p_ref[...]
    for l in range(4):
        cl = (pblk[:, (4 + l) * HP:(5 + l) * HP]
              + jnp.dot(ea_agg, wme_ref[...][:, l * HP:(l + 1) * HP],
                        preferred_element_type=jnp.float32)
              + deg * bm4_ref[l:l + 1, :] + br4_ref[l:l + 1, :])
        const_ref[l] = cl


def _k4(eap, degp, ca, p, w0e, b0e, n2g, n2b, n2a, wme, bm4, br4):
    return pl.pallas_call(
        _k4_body,
        grid=(NBLK,),
        in_specs=[pl.BlockSpec((NC, BR, HP), lambda i: (0, i, 0)),
                  pl.BlockSpec((NC, BR, 8), lambda i: (0, i, 0)),
                  pl.BlockSpec((32, 32), lambda i: (0, 0)),
                  pl.BlockSpec((BR, 128), lambda i: (i, 0)),
                  pl.BlockSpec((EF, EF), lambda i: (0, 0)),
                  pl.BlockSpec((1, EF), lambda i: (0, 0)),
                  pl.BlockSpec((1, EF), lambda i: (0, 0)),
                  pl.BlockSpec((1, EF), lambda i: (0, 0)),
                  pl.BlockSpec((1, EF), lambda i: (0, 0)),
                  pl.BlockSpec((EF, 4 * HP), lambda i: (0, 0)),
                  pl.BlockSpec((4, HP), lambda i: (0, 0)),
                  pl.BlockSpec((4, HP), lambda i: (0, 0))],
        out_specs=pl.BlockSpec((4, BR, HP), lambda i: (0, i, 0)),
        out_shape=jax.ShapeDtypeStruct((4, N, HP), jnp.float32),
    )(eap, degp, ca, p, w0e, b0e, n2g, n2b, n2a, wme, bm4, br4)


def _k5_body(sp_ref, const_ref, r_ref, pa_ref, wrr_ref, wmr_ref,
             rn_ref, y_ref):
    rn = jnp.maximum(const_ref[...]
                     + jnp.dot(r_ref[...], wrr_ref[...], preferred_element_type=jnp.float32)
                     + sp_ref[0] + sp_ref[1], 0.0)
    rn_ref[...] = rn
    y_ref[...] = pa_ref[...] + jnp.dot(rn, wmr_ref[...],
                                       preferred_element_type=jnp.float32)


def _k5(sp, const, r, pa, wrr, wmr):
    return pl.pallas_call(
        _k5_body,
        grid=(NBLK,),
        in_specs=[pl.BlockSpec((NC, BR, HP), lambda i: (0, i, 0)),
                  pl.BlockSpec((BR, HP), lambda i: (i, 0)),
                  pl.BlockSpec((BR, HP), lambda i: (i, 0)),
                  pl.BlockSpec((BR, HP), lambda i: (i, 0)),
                  pl.BlockSpec((HP, HP), lambda i: (0, 0)),
                  pl.BlockSpec((HP, HP), lambda i: (0, 0))],
        out_specs=[pl.BlockSpec((BR, HP), lambda i: (i, 0)),
                   pl.BlockSpec((BR, HP), lambda i: (i, 0))],
        out_shape=[jax.ShapeDtypeStruct((N, HP), jnp.float32),
                   jax.ShapeDtypeStruct((NPAD, HP), jnp.float32)],
    )(sp, const, r, pa, wrr, wmr)


def _k6_body(sp_ref, const_ref, r_ref, wrr_ref, w5_ref, b5_ref, probs_ref):
    r4 = jnp.maximum(const_ref[...]
                     + jnp.dot(r_ref[...], wrr_ref[...], preferred_element_type=jnp.float32)
                     + sp_ref[0] + sp_ref[1], 0.0)
    logits = jnp.dot(r4, w5_ref[...], preferred_element_type=jnp.float32) + b5_ref[...]
    a = logits[:, 0:1]
    b = logits[:, 1:2]
    mx = jnp.maximum(a, b)
    e0 = jnp.exp(a - mx)
    e1 = jnp.exp(b - mx)
    den = e0 + e1
    probs_ref[...] = jnp.concatenate(
        [e0 / den, e1 / den, jnp.zeros((BR, 6), jnp.float32)], axis=1)


def _k6(sp, const, r, wrr, w5, b5):
    return pl.pallas_call(
        _k6_body,
        grid=(NBLK,),
        in_specs=[pl.BlockSpec((NC, BR, HP), lambda i: (0, i, 0)),
                  pl.BlockSpec((BR, HP), lambda i: (i, 0)),
                  pl.BlockSpec((BR, HP), lambda i: (i, 0)),
                  pl.BlockSpec((HP, HP), lambda i: (0, 0)),
                  pl.BlockSpec((HP, HP), lambda i: (0, 0)),
                  pl.BlockSpec((1, HP), lambda i: (0, 0))],
        out_specs=pl.BlockSpec((BR, 8), lambda i: (i, 0)),
        out_shape=jax.ShapeDtypeStruct((N, 8), jnp.float32),
    )(sp, const, r, wrr, w5, b5)


def _pad(w, rows, cols):
    return jnp.zeros((rows, cols), jnp.float32).at[:w.shape[0], :w.shape[1]].set(w)


def kernel(x, z, edge_index, z1edge_index, z2edge_index, z3edge_index,
           edge_attr, pickable, l0_W, l0_b, l02_W, l02_b, l0e_W, l0e_b,
           n_gamma, n_beta, n_alpha, n2_gamma, n2_beta, n2_alpha,
           l1_Wm, l1_bm, l1_Wr, l1_br, l2_Wm, l2_bm, l2_Wr, l2_br,
           l3_Wm, l3_bm, l3_Wr, l3_br, l4_Wm, l4_bm, l4_Wr, l4_br,
           l5_W, l5_b):
    wms = [l1_Wm, l2_Wm, l3_Wm, l4_Wm]
    wrs = [l1_Wr, l2_Wr, l3_Wr, l4_Wr]
    bms = [l1_bm, l2_bm, l3_bm, l4_bm]
    brs = [l1_br, l2_br, l3_br, l4_br]

    w80 = jnp.concatenate(
        [_pad(wm[:XF], XF, HP) for wm in wms] + [_pad(wr[:XF], XF, HP) for wr in wrs],
        axis=1)
    wmr = [_pad(wm[XF:XF + H], HP, HP) for wm in wms]
    wrr = [_pad(wr[XF:], HP, HP) for wr in wrs]
    wme = jnp.concatenate([_pad(wm[XF + H:], EF, HP) for wm in wms], axis=1)
    bm4 = jnp.concatenate([_pad(b[None, :], 1, HP) for b in bms], axis=0)
    br4 = jnp.concatenate([_pad(b[None, :], 1, HP) for b in brs], axis=0)
    w02 = _pad(l02_W, XF, HP)
    b02 = _pad(l02_b[None, :], 1, HP)
    w5 = _pad(l5_W, HP, HP)
    b5 = _pad(l5_b[None, :], 1, HP)

    pad_idx = jnp.full((EP - E,), N, jnp.int32)
    src3 = jnp.concatenate([edge_index[0], pad_idx]).reshape(NW, NB, BB)
    dst3 = jnp.concatenate([edge_index[1], pad_idx]).reshape(NW, NB, BB)
    pick2 = pickable.reshape(NPB, 2, PB // 2)
    zeros_s = jnp.zeros((SRW, HP), jnp.float32)
    zeros8_s = jnp.zeros((SRW, 8), jnp.float32)
    ones_b = jnp.ones((BB, 8), jnp.float32)

    x1a, p, st = _k1(x, l0_W, l0_b[None, :], w80)
    r0, y1 = _k2(x1a, st, p, n_gamma[None, :], n_beta[None, :], n_alpha[None, :],
                 w02, b02, wmr[0])
    ca = _k3(edge_attr.reshape(E // 8, 128))
    eap, degp = _sc_pass1()(edge_attr, dst3, zeros_s, zeros8_s, ones_b)
    const = _k4(eap, degp, ca, p, l0e_W, l0e_b[None, :],
                n2_gamma[None, :], n2_beta[None, :], n2_alpha[None, :],
                wme, bm4, br4)

    r = r0
    y = y1 + eap[0, 0:1, 0:1] * 0.0
    probs = None
    for l in range(4):
        sp = _sc_segsum()(y, src3, dst3, zeros_s)
        if l < 3:
            pa = lax.slice(p, (0, (l + 1) * HP), (N, (l + 2) * HP))
            r, y = _k5(sp, const[l], r, pa, wrr[l], wmr[l + 1])
        else:
            probs = _k6(sp, const[l], r, wrr[l], w5, b5)

    picked = _sc_pick()(probs, pick2)
    return picked[:, 0:2]
